# Initial kernel scaffold; baseline (speedup 1.0000x reference)
import jax
import jax.numpy as jnp
from jax.experimental import pallas as pl


def kernel(x_day, x_week, daytime_table, weekday_table):
    raise NotImplementedError("write your pallas kernel here")


# speedup vs baseline: 9.6293x; 9.6293x over previous
import functools

import jax
import jax.numpy as jnp
from jax import lax
from jax.experimental import pallas as pl
from jax.experimental.pallas import tpu as pltpu
from jax.experimental.pallas import tpu_sc as plsc

MINUTE = 289
WEEK = 8
D = 64
N = 16384 * 50
NW = 32
PW = N // NW
C = 128
NCH = PW // C
FUSED = MINUTE * WEEK


def _fuse_tables(day, week):
    def body(day_ref, week_ref, out_ref):
        out_ref[...] = week_ref[...][:, None, :] + day_ref[...][None, :, :]

    return pl.pallas_call(
        body,
        out_shape=jax.ShapeDtypeStruct((WEEK, MINUTE, D), jnp.float32),
    )(day, week)


def _sc_body(fused_hbm, xd_hbm, xw_hbm, out_hbm, idxd, idxw, idxf, rows, gsem):
    wid = lax.axis_index("s") * 2 + lax.axis_index("c")
    rbase = wid * NCH
    obase = wid * PW

    pltpu.sync_copy(xd_hbm.at[pl.ds(rbase, NCH)], idxd)
    pltpu.sync_copy(xw_hbm.at[pl.ds(rbase, NCH)], idxw)

    def fuse(j, carry):
        for t in range(C // 16):
            s = pl.ds(t * 16, 16)
            idxf[j, s] = idxw[j, s] * MINUTE + idxd[j, s]
        return carry

    lax.fori_loop(0, NCH, fuse, 0)

    def chunk(j, carry):
        pltpu.async_copy(fused_hbm.at[idxf.at[j]], rows, gsem).wait()
        pltpu.sync_copy(rows, out_hbm.at[pl.ds(obase + j * C, C)])
        return carry

    lax.fori_loop(0, NCH, chunk, 0)


def _sc_gather(fused, xd2, xw2):
    mesh = plsc.VectorSubcoreMesh(core_axis_name="c", subcore_axis_name="s")
    run = functools.partial(
        pl.kernel,
        mesh=mesh,
        compiler_params=pltpu.CompilerParams(use_tc_tiling_on_sc=False),
        out_type=jax.ShapeDtypeStruct((N, D), jnp.float32),
        scratch_types=[
            pltpu.VMEM((NCH, C), jnp.int32),
            pltpu.VMEM((NCH, C), jnp.int32),
            pltpu.VMEM((NCH, C), jnp.int32),
            pltpu.VMEM((C, D), jnp.float32),
            pltpu.SemaphoreType.DMA,
        ],
    )(_sc_body)
    return run(fused, xd2, xw2)


@jax.jit
def kernel(x_day, x_week, daytime_table, weekday_table):
    fused = _fuse_tables(daytime_table, weekday_table).reshape(FUSED, D)
    xd2 = x_day.reshape(N // C, C)
    xw2 = x_week.reshape(N // C, C)
    out = _sc_gather(fused, xd2, xw2)
    return out.reshape(x_day.shape[0], x_day.shape[1], D)


# speedup vs baseline: 10.8591x; 1.1277x over previous
import functools

import jax
import jax.numpy as jnp
from jax import lax
from jax.experimental import pallas as pl
from jax.experimental.pallas import tpu as pltpu
from jax.experimental.pallas import tpu_sc as plsc

MINUTE = 289
WEEK = 8
D = 64
N = 16384 * 50
NW = 32
PW = N // NW
C = 128
NCH = PW // C
FUSED = MINUTE * WEEK


def _fuse_tables(day, week):
    def body(day_ref, week_ref, out_ref):
        out_ref[...] = week_ref[...][:, None, :] + day_ref[...][None, :, :]

    return pl.pallas_call(
        body,
        out_shape=jax.ShapeDtypeStruct((WEEK, MINUTE, D), jnp.float32),
    )(day, week)


NB = 4
SK = 2
NBLK = NCH // NB


def _sc_body(fused_hbm, xd_hbm, xw_hbm, out_hbm, idxd, idxw, rows,
             g0, g1, g2, g3, s0, s1, s2, s3):
    gsems = (g0, g1, g2, g3)
    ssems = (s0, s1, s2, s3)
    wid = lax.axis_index("s") * 2 + lax.axis_index("c")
    rbase = wid * NCH
    obase = wid * PW

    pltpu.sync_copy(xd_hbm.at[pl.ds(rbase, NCH)], idxd)
    pltpu.sync_copy(xw_hbm.at[pl.ds(rbase, NCH)], idxw)

    def fuse(j, carry):
        for t in range(C // 16):
            s = pl.ds(t * 16, 16)
            idxd[j, s] = idxw[j, s] * MINUTE + idxd[j, s]
        return carry

    lax.fori_loop(0, NCH, fuse, 0)

    def gather_start(b, c):
        pltpu.async_copy(fused_hbm.at[idxd.at[c]], rows.at[b], gsems[b])

    def gather_wait(b, c):
        pltpu.make_async_copy(fused_hbm.at[idxd.at[c]], rows.at[b],
                              gsems[b]).wait()

    def scatter_start(b, c):
        pltpu.async_copy(rows.at[b], out_hbm.at[pl.ds(obase + c * C, C)],
                         ssems[b])

    def scatter_wait(b):
        pltpu.make_async_copy(rows.at[b], out_hbm.at[pl.ds(obase, C)],
                              ssems[b]).wait()

    def blk(k, carry):
        for t in range(NB):
            j = k * NB + t

            @pl.when(k >= 1)
            def _():
                scatter_wait(t)

            @pl.when(k < NBLK)
            def _():
                gather_start(t, j)

            bb = (t - SK) % NB
            c = j - SK
            guard = (k < NBLK) if t >= SK else (k >= 1)

            @pl.when(guard)
            def _():
                gather_wait(bb, c)
                scatter_start(bb, c)
        return carry

    lax.fori_loop(0, NBLK + 1, blk, 0)


def _sc_gather(fused, xd2, xw2):
    mesh = plsc.VectorSubcoreMesh(core_axis_name="c", subcore_axis_name="s")
    run = functools.partial(
        pl.kernel,
        mesh=mesh,
        compiler_params=pltpu.CompilerParams(use_tc_tiling_on_sc=False),
        out_type=jax.ShapeDtypeStruct((N, D), jnp.float32),
        scratch_types=[
            pltpu.VMEM((NCH, C), jnp.int32),
            pltpu.VMEM((NCH, C), jnp.int32),
            pltpu.VMEM((NB, C, D), jnp.float32),
        ] + [pltpu.SemaphoreType.DMA] * (2 * NB),
    )(_sc_body)
    return run(fused, xd2, xw2)


@jax.jit
def kernel(x_day, x_week, daytime_table, weekday_table):
    fused = _fuse_tables(daytime_table, weekday_table).reshape(FUSED, D)
    xd2 = x_day.reshape(N // C, C)
    xw2 = x_week.reshape(N // C, C)
    out = _sc_gather(fused, xd2, xw2)
    return out.reshape(x_day.shape[0], x_day.shape[1], D)


# speedup vs baseline: 10.9383x; 1.0073x over previous
import functools

import jax
import jax.numpy as jnp
from jax import lax
from jax.experimental import pallas as pl
from jax.experimental.pallas import tpu as pltpu
from jax.experimental.pallas import tpu_sc as plsc

MINUTE = 289
WEEK = 8
D = 64
N = 16384 * 50
NW = 32
PW = N // NW
C = 128
NCH = PW // C
FUSED = MINUTE * WEEK


def _fuse_tables(day, week):
    def body(day_ref, week_ref, out_ref):
        out_ref[...] = week_ref[...][:, None, :] + day_ref[...][None, :, :]

    return pl.pallas_call(
        body,
        out_shape=jax.ShapeDtypeStruct((WEEK, MINUTE, D), jnp.float32),
    )(day, week)


L = 50


def _fuse_idx(xd, xw):
    RIN = 2048

    def body(xd_ref, xw_ref, out_ref):
        out_ref[...] = xw_ref[...] * MINUTE + xd_ref[...]

    return pl.pallas_call(
        body,
        grid=(16384 // RIN,),
        in_specs=[
            pl.BlockSpec((RIN, L), lambda i: (i, 0)),
            pl.BlockSpec((RIN, L), lambda i: (i, 0)),
        ],
        out_specs=pl.BlockSpec((RIN, L), lambda i: (i, 0)),
        out_shape=jax.ShapeDtypeStruct((16384, L), jnp.int32),
    )(xd, xw)


NB = 8
SK = 3
NBLK = NCH // NB


def _sc_body(fused_hbm, idx_hbm, out_hbm, idxf, rows, *sems):
    gsems = sems[:NB]
    ssems = sems[NB:]
    wid = lax.axis_index("s") * 2 + lax.axis_index("c")
    rbase = wid * NCH
    obase = wid * PW

    pltpu.sync_copy(idx_hbm.at[pl.ds(rbase, NCH)], idxf)

    def gather_start(b, c):
        pltpu.async_copy(fused_hbm.at[idxf.at[c]], rows.at[b], gsems[b])

    def gather_wait(b, c):
        pltpu.make_async_copy(fused_hbm.at[idxf.at[c]], rows.at[b],
                              gsems[b]).wait()

    def scatter_start(b, c):
        pltpu.async_copy(rows.at[b], out_hbm.at[pl.ds(obase + c * C, C)],
                         ssems[b])

    def scatter_wait(b):
        pltpu.make_async_copy(rows.at[b], out_hbm.at[pl.ds(obase, C)],
                              ssems[b]).wait()

    def blk(k, carry):
        for t in range(NB):
            j = k * NB + t

            @pl.when(k >= 1)
            def _():
                scatter_wait(t)

            @pl.when(k < NBLK)
            def _():
                gather_start(t, j)

            bb = (t - SK) % NB
            c = j - SK
            guard = (k < NBLK) if t >= SK else (k >= 1)

            @pl.when(guard)
            def _():
                gather_wait(bb, c)
                scatter_start(bb, c)
        return carry

    lax.fori_loop(0, NBLK + 1, blk, 0)


def _sc_gather(fused, idx2):
    mesh = plsc.VectorSubcoreMesh(core_axis_name="c", subcore_axis_name="s")
    run = functools.partial(
        pl.kernel,
        mesh=mesh,
        compiler_params=pltpu.CompilerParams(use_tc_tiling_on_sc=False),
        out_type=jax.ShapeDtypeStruct((N, D), jnp.float32),
        scratch_types=[
            pltpu.VMEM((NCH, C), jnp.int32),
            pltpu.VMEM((NB, C, D), jnp.float32),
        ] + [pltpu.SemaphoreType.DMA] * (2 * NB),
    )(_sc_body)
    return run(fused, idx2)


@jax.jit
def kernel(x_day, x_week, daytime_table, weekday_table):
    fused = _fuse_tables(daytime_table, weekday_table).reshape(FUSED, D)
    idx2 = _fuse_idx(x_day, x_week).reshape(N // C, C)
    out = _sc_gather(fused, idx2)
    return out.reshape(x_day.shape[0], x_day.shape[1], D)
